# Initial kernel scaffold; baseline (speedup 1.0000x reference)
import jax
import jax.numpy as jnp
from jax.experimental import pallas as pl


def kernel(x, edge_index, W, attn_l, attn_r):
    raise NotImplementedError("write your pallas kernel here")


# speedup vs baseline: 16.2058x; 16.2058x over previous
import functools

import jax
import jax.numpy as jnp
from jax import lax
from jax.experimental import pallas as pl
from jax.experimental.pallas import tpu as pltpu
from jax.experimental.pallas import tpu_sc as plsc

N = 10000
E = 320000
F = 128
FE = 144
NPAD = 10112
NC = 2
NS = 16
NW = NC * NS
CHUNK = 128
EPT = 10112
EPAD = EPT * NW
RBLK = 632
CBLK = 80


def _proj_body(x_ref, w_ref, al_ref, ar_ref, hext_ref, el_ref, er_ref):
    h = lax.dot_general(x_ref[...], w_ref[...],
                        (((1,), (1,)), ((), ())),
                        preferred_element_type=jnp.float32)
    lane16 = lax.broadcasted_iota(jnp.int32, (RBLK, FE - F), 1)
    extra = jnp.where(lane16 == 0, 1.0, 0.0).astype(jnp.float32)
    hext_ref[...] = jnp.concatenate([h, extra], axis=1)
    el_ref[0, 0, :] = jnp.sum(h * al_ref[...], axis=1)
    er_ref[0, 0, :] = jnp.sum(h * ar_ref[...], axis=1)


_proj = pl.pallas_call(
    _proj_body,
    grid=(NPAD // RBLK,),
    in_specs=[
        pl.BlockSpec((RBLK, F), lambda i: (i, 0)),
        pl.BlockSpec((F, F), lambda i: (0, 0)),
        pl.BlockSpec((1, F), lambda i: (0, 0)),
        pl.BlockSpec((1, F), lambda i: (0, 0)),
    ],
    out_specs=[
        pl.BlockSpec((RBLK, FE), lambda i: (i, 0)),
        pl.BlockSpec((1, 1, RBLK), lambda i: (i, 0, 0)),
        pl.BlockSpec((1, 1, RBLK), lambda i: (i, 0, 0)),
    ],
    out_shape=[
        jax.ShapeDtypeStruct((NPAD, FE), jnp.float32),
        jax.ShapeDtypeStruct((NPAD // RBLK, 1, RBLK), jnp.float32),
        jax.ShapeDtypeStruct((NPAD // RBLK, 1, RBLK), jnp.float32),
    ],
)


def _edge_body(hext_hbm, el_hbm, er_hbm, src_hbm, dst_hbm, acc_hbm,
               el_v, er_v, src_v, dst_v, w_v, rows_v, acc_sh, sem):
    cid = lax.axis_index("c")
    sid = lax.axis_index("s")
    wid = sid * NC + cid

    pltpu.sync_copy(el_hbm, el_v)
    pltpu.sync_copy(er_hbm, er_v)

    def _zero_row(r, carry):
        for j in range(FE // 16):
            rows_v[r, pl.ds(j * 16, 16)] = jnp.zeros((16,), jnp.float32)
        return carry
    lax.fori_loop(0, CHUNK, _zero_row, 0)
    for k in range(RBLK // CHUNK):
        pltpu.sync_copy(rows_v, acc_sh.at[pl.ds(sid * RBLK + k * CHUNK, CHUNK)])
    rem = RBLK % CHUNK
    if rem:
        pltpu.sync_copy(rows_v.at[pl.ds(0, rem)],
                        acc_sh.at[pl.ds(sid * RBLK + (RBLK // CHUNK) * CHUNK, rem)])
    plsc.subcore_barrier()

    def _chunk(g, carry):
        base = wid * EPT + g * CHUNK
        pltpu.sync_copy(src_hbm.at[pl.ds(base, CHUNK)], src_v)
        pltpu.sync_copy(dst_hbm.at[pl.ds(base, CHUNK)], dst_v)
        for i in range(CHUNK // 16):
            s_idx = src_v[pl.ds(i * 16, 16)]
            d_idx = dst_v[pl.ds(i * 16, 16)]
            e = plsc.load_gather(el_v, [s_idx]) + plsc.load_gather(er_v, [d_idx])
            e = jnp.where(e > 0, e, 0.2 * e)
            w_v[pl.ds(i * 16, 16)] = jnp.exp(e)
        pltpu.async_copy(hext_hbm.at[src_v], rows_v, sem).wait()
        def _scale_row(r, carry2):
            wr = jnp.zeros((16,), jnp.int32) + r
            wvec = plsc.load_gather(w_v, [wr])
            for j in range(FE // 16):
                rows_v[r, pl.ds(j * 16, 16)] = rows_v[r, pl.ds(j * 16, 16)] * wvec
            return carry2
        lax.fori_loop(0, CHUNK, _scale_row, 0)
        pltpu.sync_copy(rows_v, acc_sh.at[dst_v], add=True)
        return carry

    lax.fori_loop(0, EPT // CHUNK, _chunk, 0)

    plsc.subcore_barrier()
    pltpu.sync_copy(acc_sh.at[pl.ds(sid * RBLK, RBLK)],
                    acc_hbm.at[cid, pl.ds(sid * RBLK, RBLK)])


_edge = functools.partial(
    pl.kernel,
    out_type=jax.ShapeDtypeStruct((NC, NPAD, FE), jnp.float32),
    mesh=plsc.VectorSubcoreMesh(core_axis_name="c", subcore_axis_name="s"),
    scratch_types=[
        pltpu.VMEM((NPAD,), jnp.float32),
        pltpu.VMEM((NPAD,), jnp.float32),
        pltpu.VMEM((CHUNK,), jnp.int32),
        pltpu.VMEM((CHUNK,), jnp.int32),
        pltpu.VMEM((CHUNK,), jnp.float32),
        pltpu.VMEM((CHUNK, FE), jnp.float32),
        pltpu.VMEM_SHARED((NPAD, FE), jnp.float32),
        pltpu.SemaphoreType.DMA,
    ],
    compiler_params=pltpu.CompilerParams(needs_layout_passes=False,
                                         use_tc_tiling_on_sc=False),
)(_edge_body)


def _combine_body(a0_ref, a1_ref, out_ref):
    s = a0_ref[...] + a1_ref[...]
    lane = lax.broadcasted_iota(jnp.int32, (CBLK, FE), 1)
    den = jnp.sum(jnp.where(lane == F, s, 0.0), axis=1, keepdims=True)
    msg = s[:, :F]
    out_ref[...] = jnp.where(den > 0, msg / den, 0.0)


_combine = pl.pallas_call(
    _combine_body,
    grid=(N // CBLK,),
    in_specs=[
        pl.BlockSpec((CBLK, FE), lambda i: (i, 0)),
        pl.BlockSpec((CBLK, FE), lambda i: (i, 0)),
    ],
    out_specs=pl.BlockSpec((CBLK, F), lambda i: (i, 0)),
    out_shape=jax.ShapeDtypeStruct((N, F), jnp.float32),
)


def kernel(x, edge_index, W, attn_l, attn_r):
    x_pad = jnp.zeros((NPAD, F), jnp.float32).at[:N].set(x)
    al = attn_l.reshape(1, F)
    ar = attn_r.reshape(1, F)
    hext, el3, er3 = _proj(x_pad, W, al, ar)
    el = el3.reshape(NPAD)
    er = er3.reshape(NPAD)
    pad = EPAD - E
    src_p = jnp.concatenate([edge_index[0], jnp.zeros((pad,), jnp.int32)])
    dst_p = jnp.concatenate([edge_index[1], jnp.full((pad,), N, jnp.int32)])
    acc = _edge(hext, el, er, src_p, dst_p)
    return _combine(acc[0], acc[1])
